# baseline (speedup 1.0000x reference)
import functools

import jax
import jax.numpy as jnp
from jax import lax
from jax.experimental import pallas as pl
from jax.experimental.pallas import tpu as pltpu
from jax.experimental.pallas import tpu_sc as plsc

N_CORES = 2
N_SUBCORES = 16
LANES = 16
N_WORKERS = N_CORES * N_SUBCORES

BATCH = 16384
D = 64
PER_W = BATCH // N_WORKERS
CH = 256
N_CH = PER_W // CH
IDX_ROWS = PER_W // 128


def _sc_body(uidx_hbm, iidx_hbm, uf_hbm, if_hbm, ub_hbm, ib_hbm,
             out_hbm, uidx_v, iidx_v, utid_v, itid_v, urows_v, irows_v,
             ubias_v, ibias_v, out_v, sem_u, sem_i, sem_b):
    wid = lax.axis_index("s") * N_CORES + lax.axis_index("c")

    pltpu.sync_copy(uidx_hbm.at[pl.ds(wid, 1)], uidx_v)
    pltpu.sync_copy(iidx_hbm.at[pl.ds(wid, 1)], iidx_v)

    lane = lax.iota(jnp.int32, LANES)
    zeros = lane * 0

    def build(g, carry):
        s = pl.ds(g * LANES, LANES)
        uvals = uidx_v[0, s]
        ivals = iidx_v[0, s]
        j = g // 8
        k = pl.ds((g % 8) * LANES, LANES)
        utid_v[j, k] = jnp.where(uvals >= HALF, uvals - HALF, uvals)
        itid_v[j, k] = jnp.where(ivals >= HALF, ivals - HALF, ivals)
        return carry

    lax.fori_loop(0, PER_W // LANES, build, 0)

    def fire_bias(g, carry):
        uvals = uidx_v[0, pl.ds(g * LANES, LANES)]
        ivals = iidx_v[0, pl.ds(g * LANES, LANES)]
        for p in range(LANES):
            su = uvals[p]
            si = ivals[p]
            su_al = pl.multiple_of(su & ~7, 8)
            si_al = pl.multiple_of(si & ~7, 8)
            pb = pl.multiple_of((g * LANES + p) * 8, 8)
            pltpu.make_async_copy(
                ub_hbm.at[pl.ds(su_al, 8)], ubias_v.at[pl.ds(pb, 8)],
                sem_b).start()
            pltpu.make_async_copy(
                ib_hbm.at[pl.ds(si_al, 8)], ibias_v.at[pl.ds(pb, 8)],
                sem_b).start()
        return carry

    lax.fori_loop(0, PER_W // LANES, fire_bias, 0)

    for c in range(N_CH):
        cps = []
        for j in range(CH // 128):
            jj = c * (CH // 128) + j
            rows = pl.ds(j * 128, 128)
            cps.append(pltpu.async_copy(
                uf_hbm.at[utid_v.at[jj]], urows_v.at[rows], sem_u))
            cps.append(pltpu.async_copy(
                if_hbm.at[itid_v.at[jj]], irows_v.at[rows], sem_i))
        for cp in cps:
            cp.wait()

        for g in range(CH // LANES):
            p_loc = g * LANES + lane
            rows16 = c * CH + p_loc
            uvals = uidx_v[0, pl.ds(c * CH + g * LANES, LANES)]
            ivals = iidx_v[0, pl.ds(c * CH + g * LANES, LANES)]
            uoff = jnp.where(uvals >= HALF, D, 0)
            ioff = jnp.where(ivals >= HALF, D, 0)
            acc = plsc.load_gather(ubias_v, [rows16 * 8 + (uvals & 7)])
            acc = acc + plsc.load_gather(ibias_v, [rows16 * 8 + (ivals & 7)])
            for j in range(D):
                u = plsc.load_gather(urows_v, [p_loc, uoff + j])
                v = plsc.load_gather(irows_v, [p_loc, ioff + j])
                acc = acc + u * v
            acc = 1.0 / (1.0 + jnp.exp(-acc))
            out_v[pl.ds(c * CH + g * LANES, LANES)] = acc

    pltpu.sync_copy(out_v, out_hbm.at[pl.ds(wid * PER_W, PER_W)])


HALF = 500000
R_BLK = 2000
N_BLK = HALF // R_BLK


def _repack_body(a_ref, b_ref, o_ref):
    o_ref[:, 0:D] = a_ref[...]
    o_ref[:, D:2 * D] = b_ref[...]


@jax.jit
def _tc_repack(x):
    return pl.pallas_call(
        _repack_body,
        grid=(N_BLK,),
        in_specs=[
            pl.BlockSpec((R_BLK, D), lambda i: (i, 0)),
            pl.BlockSpec((R_BLK, D), lambda i: (i + N_BLK, 0)),
        ],
        out_specs=pl.BlockSpec((R_BLK, 2 * D), lambda i: (i, 0)),
        out_shape=jax.ShapeDtypeStruct((HALF, 2 * D), jnp.float32),
        compiler_params=pltpu.CompilerParams(
            dimension_semantics=("arbitrary",)),
    )(x, x)


@jax.jit
def _baseline_cf_sc(uidx, iidx, uf2, if2, user_bias, item_bias):
    mesh = plsc.VectorSubcoreMesh(core_axis_name="c", subcore_axis_name="s")
    run = functools.partial(
        pl.kernel,
        mesh=mesh,
        compiler_params=pltpu.CompilerParams(needs_layout_passes=False),
        out_type=jax.ShapeDtypeStruct((BATCH,), jnp.float32),
        scratch_types=[
            pltpu.VMEM((1, PER_W), jnp.int32),
            pltpu.VMEM((1, PER_W), jnp.int32),
            pltpu.VMEM((IDX_ROWS, 128), jnp.int32),
            pltpu.VMEM((IDX_ROWS, 128), jnp.int32),
            pltpu.VMEM((CH, 128), jnp.float32),
            pltpu.VMEM((CH, 128), jnp.float32),
            pltpu.VMEM((PER_W * 8,), jnp.float32),
            pltpu.VMEM((PER_W * 8,), jnp.float32),
            pltpu.VMEM((PER_W,), jnp.float32),
            pltpu.SemaphoreType.DMA,
            pltpu.SemaphoreType.DMA,
            pltpu.SemaphoreType.DMA,
        ],
    )(_sc_body)
    return run(uidx, iidx, uf2, if2, user_bias, item_bias)


def kernel(data, user_factors, item_factors, user_bias, item_bias):
    uidx = data[:, 0].reshape(N_WORKERS, PER_W)
    iidx = data[:, 1].reshape(N_WORKERS, PER_W)
    uf2 = _tc_repack(user_factors)
    if2 = _tc_repack(item_factors)
    out = _baseline_cf_sc(uidx, iidx, uf2, if2,
                          user_bias.reshape(-1), item_bias.reshape(-1))
    return out.reshape(BATCH, 1)


# speedup vs baseline: 2.6699x; 2.6699x over previous
import functools

import jax
import jax.numpy as jnp
from jax import lax
from jax.experimental import pallas as pl
from jax.experimental.pallas import tpu as pltpu
from jax.experimental.pallas import tpu_sc as plsc

N_CORES = 2
N_SUBCORES = 16
LANES = 16
N_WORKERS = N_CORES * N_SUBCORES

BATCH = 16384
D = 64
PER_W = BATCH // N_WORKERS
CH = 32
N_CH = PER_W // CH
G_PER_CH = CH // LANES


def _sc_body(uidx_hbm, iidx_hbm, uf_hbm, if_hbm, ub_hbm, ib_hbm, dummy_hbm,
             out_hbm, uidx_v, iidx_v, utile_v, itile_v,
             ubias_v, ibias_v, out_v, sem_u, sem_i, sem_b):
    wid = lax.axis_index("s") * N_CORES + lax.axis_index("c")

    pltpu.sync_copy(uidx_hbm.at[pl.ds(wid, 1)], uidx_v)
    pltpu.sync_copy(iidx_hbm.at[pl.ds(wid, 1)], iidx_v)

    def fire_bias(g, carry):
        uvals = uidx_v[0, pl.ds(g * LANES, LANES)]
        ivals = iidx_v[0, pl.ds(g * LANES, LANES)]
        for p in range(LANES):
            su = uvals[p]
            si = ivals[p]
            su_al = pl.multiple_of(su & ~7, 8)
            si_al = pl.multiple_of(si & ~7, 8)
            pb = pl.multiple_of((g * LANES + p) * 8, 8)
            pltpu.make_async_copy(
                ub_hbm.at[pl.ds(su_al, 8)], ubias_v.at[pl.ds(pb, 8)],
                sem_b).start()
            pltpu.make_async_copy(
                ib_hbm.at[pl.ds(si_al, 8)], ibias_v.at[pl.ds(pb, 8)],
                sem_b).start()
        return carry

    lax.fori_loop(0, PER_W // LANES, fire_bias, 0)
    pltpu.make_async_copy(ub_hbm.at[pl.ds(0, PER_W * 8)], ubias_v,
                          sem_b).wait()
    pltpu.make_async_copy(ib_hbm.at[pl.ds(0, PER_W * 8)], ibias_v,
                          sem_b).wait()

    lane = lax.iota(jnp.int32, LANES)
    zeros = lane * 0

    def chunk(c, carry):
        for g in range(G_PER_CH):
            uvals = uidx_v[0, pl.ds(c * CH + g * LANES, LANES)]
            ivals = iidx_v[0, pl.ds(c * CH + g * LANES, LANES)]
            for p in range(LANES):
                su = uvals[p]
                si = ivals[p]
                pltpu.make_async_copy(
                    uf_hbm.at[pl.ds(su >> 3, 1)],
                    utile_v.at[pl.ds(g * LANES + p, 1)], sem_u).start()
                pltpu.make_async_copy(
                    if_hbm.at[pl.ds(si >> 3, 1)],
                    itile_v.at[pl.ds(g * LANES + p, 1)], sem_i).start()
        pltpu.make_async_copy(dummy_hbm, utile_v, sem_u).wait()
        pltpu.make_async_copy(dummy_hbm, itile_v, sem_i).wait()

        for g in range(G_PER_CH):
            p_loc = g * LANES + lane
            rows16 = c * CH + p_loc
            uvals = uidx_v[0, pl.ds(c * CH + g * LANES, LANES)]
            ivals = iidx_v[0, pl.ds(c * CH + g * LANES, LANES)]
            urow = uvals & 7
            irow = ivals & 7
            acc = plsc.load_gather(ubias_v, [rows16 * 8 + urow])
            acc = acc + plsc.load_gather(ibias_v, [rows16 * 8 + irow])
            for j in range(D):
                colj = zeros + j
                u = plsc.load_gather(utile_v, [p_loc, urow, colj])
                v = plsc.load_gather(itile_v, [p_loc, irow, colj])
                acc = acc + u * v
            acc = 1.0 / (1.0 + jnp.exp(-acc))
            out_v[pl.ds(c * CH + g * LANES, LANES)] = acc
        return carry

    lax.fori_loop(0, N_CH, chunk, 0)
    pltpu.sync_copy(out_v, out_hbm.at[pl.ds(wid * PER_W, PER_W)])


@jax.jit
def _baseline_cf_sc(uidx, iidx, uf3, if3, user_bias, item_bias, dummy):
    mesh = plsc.VectorSubcoreMesh(core_axis_name="c", subcore_axis_name="s")
    run = functools.partial(
        pl.kernel,
        mesh=mesh,
        compiler_params=pltpu.CompilerParams(needs_layout_passes=False),
        out_type=jax.ShapeDtypeStruct((BATCH,), jnp.float32),
        scratch_types=[
            pltpu.VMEM((1, PER_W), jnp.int32),
            pltpu.VMEM((1, PER_W), jnp.int32),
            pltpu.VMEM((CH, 8, D), jnp.float32),
            pltpu.VMEM((CH, 8, D), jnp.float32),
            pltpu.VMEM((PER_W * 8,), jnp.float32),
            pltpu.VMEM((PER_W * 8,), jnp.float32),
            pltpu.VMEM((PER_W,), jnp.float32),
            pltpu.SemaphoreType.DMA,
            pltpu.SemaphoreType.DMA,
            pltpu.SemaphoreType.DMA,
        ],
    )(_sc_body)
    return run(uidx, iidx, uf3, if3, user_bias, item_bias, dummy)


def kernel(data, user_factors, item_factors, user_bias, item_bias):
    uidx = data[:, 0].reshape(N_WORKERS, PER_W)
    iidx = data[:, 1].reshape(N_WORKERS, PER_W)
    uf3 = user_factors.reshape(125000, 8, D)
    if3 = item_factors.reshape(125000, 8, D)
    dummy = jnp.zeros((CH, 8, D), jnp.float32)
    out = _baseline_cf_sc(uidx, iidx, uf3, if3,
                          user_bias.reshape(-1), item_bias.reshape(-1), dummy)
    return out.reshape(BATCH, 1)
